# baseline (speedup 1.0000x reference)
import numpy as np
from itertools import combinations

import jax
import jax.numpy as jnp
from jax import lax
from jax.experimental import pallas as pl
from jax.experimental.pallas import tpu as pltpu
from jax.experimental.pallas import tpu_sc as plsc

_SIZE_IN = 14
_HOTNESS = 3
_BATCH = 4096
_IN_DIM = 2 ** _SIZE_IN
_NSEL = 364

_FILT = np.array([sum(2 ** i for i in c) for c in combinations(range(_SIZE_IN), _HOTNESS)],
                 dtype=np.int32)
_GRAN = np.unique(_FILT >> 4)

_RUNS = []
_packed = {}
_off = 0
_start = _prev = int(_GRAN[0])
for _g in list(map(int, _GRAN[1:])) + [int(_GRAN[-1]) + 2]:
    if _g == _prev + 1:
        _prev = _g
        continue
    _w = 16 * (_prev - _start + 1)
    _RUNS.append((16 * _start, _w, _off))
    for _k in range(_start, _prev + 1):
        _packed[_k] = _off + 16 * (_k - _start)
    _off += _w
    _start = _prev = _g
_PACKED_W = _off

_WORDPOS = np.array([_packed[int(f) >> 4] + (int(f) & 15) for f in _FILT],
                    dtype=np.int32)

_NVEC = 23
_OFFS = [16 * v for v in range(_NVEC - 1)] + [_NSEL - 16]
_WP = np.stack([_WORDPOS[o:o + 16] for o in _OFFS]).astype(np.int32)

_NCORES = 2
_NSUB = 16
_NTILES = _NCORES * _NSUB
_ROWS_PER_TILE = _BATCH // _NTILES
_BC = 16
_NCHUNK = _ROWS_PER_TILE // _BC


def _body(in_hbm, wp_hbm, out_hbm, gbuf, wp_v, obuf, sems):
    wid = lax.axis_index("s") * _NCORES + lax.axis_index("c")
    pltpu.sync_copy(wp_hbm, wp_v)

    def fire(c, buf):
        b0 = wid * _ROWS_PER_TILE + c * _BC
        for (cs, w, po) in _RUNS:
            pltpu.make_async_copy(
                in_hbm.at[pl.ds(b0, _BC), pl.ds(cs, w)],
                gbuf.at[buf, :, pl.ds(po, w)], sems.at[buf]).start()

    def drain(buf):
        pltpu.make_async_copy(
            in_hbm.at[pl.ds(0, _BC), pl.ds(0, _PACKED_W)],
            gbuf.at[buf], sems.at[buf]).wait()

    def compact(c, buf):
        gv = gbuf.at[buf]

        def row_body(r, _):
            rsplat = jnp.zeros((16,), jnp.int32) + r
            for v in range(_NVEC):
                x = plsc.load_gather(gv, [rsplat, wp_v[v]])
                obuf[r, pl.ds(_OFFS[v], 16)] = x * 12.0 - 6.0
            return 0

        lax.fori_loop(0, _BC, row_body, 0, unroll=False)
        b0 = wid * _ROWS_PER_TILE + c * _BC
        pltpu.sync_copy(obuf, out_hbm.at[pl.ds(b0, _BC), :])

    fire(0, 0)

    def chunk_body(c, _):
        buf = lax.rem(c, 2)

        @pl.when(c + 1 < _NCHUNK)
        def _():
            fire(c + 1, 1 - buf)

        drain(buf)
        compact(c, buf)
        return 0

    lax.fori_loop(0, _NCHUNK, chunk_body, 0, unroll=False)


def kernel(input_var):
    wp = jnp.asarray(_WP)
    mesh = plsc.VectorSubcoreMesh(core_axis_name="c", subcore_axis_name="s",
                                  num_cores=_NCORES, num_subcores=_NSUB)
    out = pl.kernel(
        _body,
        out_type=jax.ShapeDtypeStruct((_BATCH, _NSEL), jnp.float32),
        mesh=mesh,
        scratch_types=[
            pltpu.VMEM((2, _BC, _PACKED_W), jnp.float32),
            pltpu.VMEM((_NVEC, 16), jnp.int32),
            pltpu.VMEM((_BC, _NSEL), jnp.float32),
            pltpu.SemaphoreType.DMA((2,)),
        ],
        compiler_params=pltpu.CompilerParams(needs_layout_passes=False,
                                             use_tc_tiling_on_sc=False),
    )(input_var, wp)
    return out


# speedup vs baseline: 2.4169x; 2.4169x over previous
import numpy as np
from itertools import combinations

import jax
import jax.numpy as jnp
from jax import lax
from jax.experimental import pallas as pl
from jax.experimental.pallas import tpu as pltpu
from jax.experimental.pallas import tpu_sc as plsc

_SIZE_IN = 14
_HOTNESS = 3
_BATCH = 4096
_IN_DIM = 2 ** _SIZE_IN
_NSEL = 364
_NBAND = _BATCH // 8
_NTILE = _IN_DIM // 128

_FILT = np.array([sum(2 ** i for i in c) for c in combinations(range(_SIZE_IN), _HOTNESS)],
                 dtype=np.int32)
_GRAN = np.unique(_FILT >> 4)

_RUNS = []
_packed = {}
_off = 0


def _flush_run(start, prev, off):
    w = 16 * (prev - start + 1)
    _RUNS.append((start >> 3, (start & 7) * 16, w, off))
    for k in range(start, prev + 1):
        _packed[k] = off + 16 * (k - start)
    return off + w


_start = _prev = int(_GRAN[0])
for _g in list(map(int, _GRAN[1:])) + [int(_GRAN[-1]) + 2]:
    if _g == _prev + 1 and (_g >> 3) == (_start >> 3):
        _prev = _g
        continue
    _off = _flush_run(_start, _prev, _off)
    _start = _prev = _g
_PACKED_W = _off

_WORDPOS = np.array([_packed[int(f) >> 4] + (int(f) & 15) for f in _FILT],
                    dtype=np.int32)

_NVEC = 23
_OFFS = [16 * v for v in range(_NVEC - 1)] + [_NSEL - 16]
_WP = np.stack([_WORDPOS[o:o + 16] for o in _OFFS]).astype(np.int32)

_NCORES = 2
_NSUB = 16
_NTILES = _NCORES * _NSUB
_ROWS_PER_TILE = _BATCH // _NTILES
_BC = 16
_NCHUNK = _ROWS_PER_TILE // _BC


def _body(in_hbm, wp_hbm, out_hbm, gbuf, wp_v, obuf, sems):
    wid = lax.axis_index("s") * _NCORES + lax.axis_index("c")
    pltpu.sync_copy(wp_hbm, wp_v)

    def fire(c, buf):
        band0 = (wid * _ROWS_PER_TILE + c * _BC) // 8
        gv = gbuf.at[buf]
        for (t, q, w, po) in _RUNS:
            pltpu.make_async_copy(
                in_hbm.at[pl.ds(band0, 2), t, :, pl.ds(q, w)],
                gv.at[:, :, pl.ds(po, w)], sems.at[buf]).start()

    def drain(buf):
        gv = gbuf.at[buf]
        for k in range(_PACKED_W // 128):
            pltpu.make_async_copy(
                in_hbm.at[pl.ds(0, 2), 0, :, pl.ds(0, 128)],
                gv.at[:, :, pl.ds(128 * k, 128)], sems.at[buf]).wait()

    def compact(c, buf):
        gv = gbuf.at[buf]

        def row_body(r, _):
            band = jnp.zeros((16,), jnp.int32) + lax.div(r, 8)
            p = jnp.zeros((16,), jnp.int32) + lax.rem(r, 8)
            for v in range(_NVEC):
                x = plsc.load_gather(gv, [band, p, wp_v[v]])
                obuf[r, pl.ds(_OFFS[v], 16)] = x * 12.0 - 6.0
            return 0

        lax.fori_loop(0, _BC, row_body, 0, unroll=False)
        b0 = wid * _ROWS_PER_TILE + c * _BC
        pltpu.sync_copy(obuf, out_hbm.at[pl.ds(b0, _BC), :])

    fire(0, 0)

    def chunk_body(c, _):
        buf = lax.rem(c, 2)

        @pl.when(c + 1 < _NCHUNK)
        def _():
            fire(c + 1, 1 - buf)

        drain(buf)
        compact(c, buf)
        return 0

    lax.fori_loop(0, _NCHUNK, chunk_body, 0, unroll=False)


def kernel(input_var):
    in4 = input_var.reshape(_NBAND, 8, _NTILE, 128).transpose(0, 2, 1, 3)
    wp = jnp.asarray(_WP)
    mesh = plsc.VectorSubcoreMesh(core_axis_name="c", subcore_axis_name="s",
                                  num_cores=_NCORES, num_subcores=_NSUB)
    out = pl.kernel(
        _body,
        out_type=jax.ShapeDtypeStruct((_BATCH, _NSEL), jnp.float32),
        mesh=mesh,
        scratch_types=[
            pltpu.VMEM((2, 2, 8, _PACKED_W), jnp.float32),
            pltpu.VMEM((_NVEC, 16), jnp.int32),
            pltpu.VMEM((_BC, _NSEL), jnp.float32),
            pltpu.SemaphoreType.DMA((2,)),
        ],
        compiler_params=pltpu.CompilerParams(needs_layout_passes=False,
                                             use_tc_tiling_on_sc=False),
    )(in4, wp)
    return out


# speedup vs baseline: 2.8042x; 1.1602x over previous
import numpy as np
from itertools import combinations

import jax
import jax.numpy as jnp
from jax import lax
from jax.experimental import pallas as pl
from jax.experimental.pallas import tpu as pltpu
from jax.experimental.pallas import tpu_sc as plsc

_SIZE_IN = 14
_HOTNESS = 3
_BATCH = 4096
_IN_DIM = 2 ** _SIZE_IN
_NSEL = 364
_NBAND = _BATCH // 8
_D = 64

_FILT = np.array([sum(2 ** i for i in c) for c in combinations(range(_SIZE_IN), _HOTNESS)],
                 dtype=np.int32)
_G64 = np.unique(_FILT >> 6)
_NG = len(_G64)

_STAT = ((_G64 >> 1) * 16 + (_G64 & 1)).astype(np.int32)

_NENT = 8 * _NG
_NENTP = ((_NENT + 15) // 16) * 16
_SIDX = np.empty((_NENTP,), np.int32)
for _p in range(8):
    _SIDX[_p * _NG:(_p + 1) * _NG] = _p * 2 + _STAT
_SIDX[_NENT:] = _SIDX[_NENT - 1]

_SLOT = {int(g): i for i, g in enumerate(_G64)}
_SROW = np.array([_SLOT[int(f) >> 6] for f in _FILT], dtype=np.int32)
_SCOL = (_FILT & 63).astype(np.int32)

_NVEC = 23
_OFFS = [16 * v for v in range(_NVEC - 1)] + [_NSEL - 16]
_WROW = np.stack([_SROW[o:o + 16] for o in _OFFS]).astype(np.int32)
_WCOL = np.stack([_SCOL[o:o + 16] for o in _OFFS]).astype(np.int32)

_NCORES = 2
_NSUB = 16
_NTILES = _NCORES * _NSUB
_BANDS_PER_TILE = _NBAND // _NTILES


def _body(tab_hbm, sidx_hbm, wrow_hbm, wcol_hbm, out_hbm,
          gbuf, idxbuf, sidx_v, wrow_v, wcol_v, obuf, sems):
    wid = lax.axis_index("s") * _NCORES + lax.axis_index("c")
    pltpu.sync_copy(sidx_hbm, sidx_v)
    pltpu.sync_copy(wrow_hbm, wrow_v)
    pltpu.sync_copy(wcol_hbm, wcol_v)

    def fire(c, buf):
        band = wid * _BANDS_PER_TILE + c
        base = band * 2048
        iv = idxbuf.at[buf]
        for k in range(_NENTP // 16):
            iv[pl.ds(16 * k, 16)] = sidx_v[pl.ds(16 * k, 16)] + base
        pltpu.make_async_copy(tab_hbm.at[iv], gbuf.at[buf],
                              sems.at[buf]).start()

    def drain(buf):
        pltpu.make_async_copy(tab_hbm.at[pl.ds(0, _NENTP), :], gbuf.at[buf],
                              sems.at[buf]).wait()

    def compact(c, buf):
        gv = gbuf.at[buf]

        def row_body(r, _):
            rbase = r * _NG
            for v in range(_NVEC):
                x = plsc.load_gather(gv, [wrow_v[v] + rbase, wcol_v[v]])
                obuf[r, pl.ds(_OFFS[v], 16)] = x * 12.0 - 6.0
            return 0

        lax.fori_loop(0, 8, row_body, 0, unroll=False)
        b0 = (wid * _BANDS_PER_TILE + c) * 8
        pltpu.sync_copy(obuf, out_hbm.at[pl.ds(b0, 8), :])

    fire(0, 0)

    def chunk_body(c, _):
        buf = lax.rem(c, 2)

        @pl.when(c + 1 < _BANDS_PER_TILE)
        def _():
            fire(c + 1, 1 - buf)

        drain(buf)
        compact(c, buf)
        return 0

    lax.fori_loop(0, _BANDS_PER_TILE, chunk_body, 0, unroll=False)


def kernel(input_var):
    tab = (input_var.reshape(_NBAND, 8, _IN_DIM // 128, 128)
           .transpose(0, 2, 1, 3)
           .reshape(_NBAND * (_IN_DIM // 128) * 8 * 2, _D))
    sidx = jnp.asarray(_SIDX)
    wrow = jnp.asarray(_WROW)
    wcol = jnp.asarray(_WCOL)
    mesh = plsc.VectorSubcoreMesh(core_axis_name="c", subcore_axis_name="s",
                                  num_cores=_NCORES, num_subcores=_NSUB)
    out = pl.kernel(
        _body,
        out_type=jax.ShapeDtypeStruct((_BATCH, _NSEL), jnp.float32),
        mesh=mesh,
        scratch_types=[
            pltpu.VMEM((2, _NENTP, _D), jnp.float32),
            pltpu.VMEM((2, _NENTP), jnp.int32),
            pltpu.VMEM((_NENTP,), jnp.int32),
            pltpu.VMEM((_NVEC, 16), jnp.int32),
            pltpu.VMEM((_NVEC, 16), jnp.int32),
            pltpu.VMEM((8, _NSEL), jnp.float32),
            pltpu.SemaphoreType.DMA((2,)),
        ],
        compiler_params=pltpu.CompilerParams(needs_layout_passes=False,
                                             use_tc_tiling_on_sc=False),
    )(tab, sidx, wrow, wcol)
    return out


# speedup vs baseline: 3.0855x; 1.1003x over previous
import numpy as np
from itertools import combinations

import jax
import jax.numpy as jnp
from jax import lax
from jax.experimental import pallas as pl
from jax.experimental.pallas import tpu as pltpu
from jax.experimental.pallas import tpu_sc as plsc

_SIZE_IN = 14
_HOTNESS = 3
_BATCH = 4096
_IN_DIM = 2 ** _SIZE_IN
_NSEL = 364
_NBAND = _BATCH // 8
_NT = _IN_DIM // 128
_D = 512

_FILT = np.array([sum(2 ** i for i in c) for c in combinations(range(_SIZE_IN), _HOTNESS)],
                 dtype=np.int32)
_TILES = np.unique(_FILT >> 7)
_NE = len(_TILES)

_SIDX = (_TILES.astype(np.int32) * 2)

_SLOT = {int(t): i for i, t in enumerate(_TILES)}
_SROW = np.array([_SLOT[int(f) >> 7] for f in _FILT], dtype=np.int32)
_SCOL = (_FILT & 127).astype(np.int32)

_NVEC = 23
_SROWP = np.concatenate([_SROW, np.full(4, _SROW[-1], np.int32)])
_SCOLP = np.concatenate([_SCOL, np.full(4, _SCOL[-1], np.int32)])
_WROW = np.stack([_SROWP[16 * v:16 * v + 16] for v in range(_NVEC)]).astype(np.int32)
_WCOL = np.stack([_SCOLP[16 * v:16 * v + 16] for v in range(_NVEC)]).astype(np.int32)

_NCORES = 2
_NSUB = 16
_NTILES = _NCORES * _NSUB
_NCHUNK = (_BATCH // _NTILES) // 4


def _body(tab_hbm, sidx_hbm, wrow_hbm, wcol_hbm, out_hbm,
          gbuf, idxbuf, sidx_v, wrow_v, wcol_v, obuf, sems):
    wid = lax.axis_index("s") * _NCORES + lax.axis_index("c")
    pltpu.sync_copy(sidx_hbm, sidx_v)
    pltpu.sync_copy(wrow_hbm, wrow_v)
    pltpu.sync_copy(wcol_hbm, wcol_v)

    def fire(c, buf):
        half = wid * _NCHUNK + c
        base = lax.div(half, 2) * 256 + lax.rem(half, 2)
        iv = idxbuf.at[buf]
        for k in range(_NE // 16):
            iv[pl.ds(16 * k, 16)] = sidx_v[pl.ds(16 * k, 16)] + base
        pltpu.make_async_copy(tab_hbm.at[iv], gbuf.at[buf],
                              sems.at[buf]).start()

    def drain(buf):
        pltpu.make_async_copy(tab_hbm.at[pl.ds(0, _NE), :], gbuf.at[buf],
                              sems.at[buf]).wait()

    def compact(c, buf):
        gv = gbuf.at[buf]

        def row_body(r, _):
            cbase = r * 128
            for v in range(_NVEC):
                x = plsc.load_gather(gv, [wrow_v[v], wcol_v[v] + cbase])
                obuf[16 * v // 128, r, pl.ds((16 * v) % 128, 16)] = x * 12.0 - 6.0
            return 0

        lax.fori_loop(0, 4, row_body, 0, unroll=False)
        half = wid * _NCHUNK + c
        band = lax.div(half, 2)
        p0 = lax.rem(half, 2) * 4
        pltpu.sync_copy(obuf, out_hbm.at[band, :, pl.ds(p0, 4), :])

    fire(0, 0)

    def chunk_body(c, _):
        buf = lax.rem(c, 2)

        @pl.when(c + 1 < _NCHUNK)
        def _():
            fire(c + 1, 1 - buf)

        drain(buf)
        compact(c, buf)
        return 0

    lax.fori_loop(0, _NCHUNK, chunk_body, 0, unroll=False)


def kernel(input_var):
    tab = (input_var.reshape(_NBAND, 8, _NT, 128)
           .transpose(0, 2, 1, 3)
           .reshape(_NBAND * _NT * 2, _D))
    sidx = jnp.asarray(_SIDX)
    wrow = jnp.asarray(_WROW)
    wcol = jnp.asarray(_WCOL)
    mesh = plsc.VectorSubcoreMesh(core_axis_name="c", subcore_axis_name="s",
                                  num_cores=_NCORES, num_subcores=_NSUB)
    out4 = pl.kernel(
        _body,
        out_type=jax.ShapeDtypeStruct((_NBAND, 3, 8, 128), jnp.float32),
        mesh=mesh,
        scratch_types=[
            pltpu.VMEM((2, _NE, _D), jnp.float32),
            pltpu.VMEM((2, _NE), jnp.int32),
            pltpu.VMEM((_NE,), jnp.int32),
            pltpu.VMEM((_NVEC, 16), jnp.int32),
            pltpu.VMEM((_NVEC, 16), jnp.int32),
            pltpu.VMEM((3, 4, 128), jnp.float32),
            pltpu.SemaphoreType.DMA((2,)),
        ],
        compiler_params=pltpu.CompilerParams(needs_layout_passes=False,
                                             use_tc_tiling_on_sc=False),
    )(tab, sidx, wrow, wcol)
    return out4.transpose(0, 2, 1, 3).reshape(_BATCH, 384)[:, :_NSEL]
